# baseline (speedup 1.0000x reference)
import jax
import jax.numpy as jnp
from jax import lax
from jax.experimental import pallas as pl
from jax.experimental.pallas import tpu as pltpu
from jax.experimental.pallas import tpu_sc as plsc

CLS = 1_000_000
FEAT = 32
B = 16384
NW = 32
ROWS = B // NW
L = 16
NG = ROWS // L


def _body(xsT_hbm, ys_hbm, ct_hbm, out_hbm,
          idx2, hys, colF, xv, cntf, zbuf, idxb, obuf, sem, hsem, histo):
    cid = lax.axis_index("c")
    sid = lax.axis_index("s")
    wid = sid * 2 + cid

    pltpu.sync_copy(ys_hbm.at[pl.ds(wid * 4, 4)], idx2)
    pltpu.sync_copy(ys_hbm.at[pl.ds(sid * 8, 8)], hys)

    xcopy = pltpu.async_copy(xsT_hbm.at[:, pl.ds(wid * ROWS, ROWS)], xv, hsem)

    for i in range(8):
        zbuf[pl.ds(i * L, L)] = jnp.zeros((L,), jnp.int32)
    for j in range(8):
        pltpu.sync_copy(zbuf, histo.at[hys.at[j]])
    plsc.subcore_barrier()

    for i in range(8):
        zbuf[pl.ds(i * L, L)] = jnp.ones((L,), jnp.int32)
    for j in range(8):
        pltpu.sync_copy(zbuf, histo.at[hys.at[j]], add=True)
    plsc.subcore_barrier()

    for j in range(4):
        pltpu.sync_copy(histo.at[idx2.at[j]], cntf.at[pl.ds(j * 128, 128)])

    xcopy.wait()

    def group(g, acc):
        yv = idx2[g // 8, pl.ds((g % 8) * L, L)]
        for f in range(FEAT):
            idxb[f // 8, pl.ds((f % 8) * L, L)] = yv + f * CLS
        copies = [
            pltpu.async_copy(ct_hbm.at[idxb.at[j]], colF.at[g, j], sem)
            for j in range(4)
        ]
        for c in copies:
            c.wait()

        ci = cntf[pl.ds(g * L, L)]
        w = 1.0 / ci.astype(jnp.float32)
        gacc = jnp.zeros((L,), jnp.float32)
        for f in range(FEAT):
            xf = xv[f, pl.ds(g * L, L)]
            cf = colF[g, f // 8, pl.ds((f % 8) * L, L)]
            d = xf - cf
            gacc = gacc + d * d
        return acc + gacc * w

    acc = lax.fori_loop(0, NG, group, jnp.zeros((L,), jnp.float32))
    obuf[...] = acc
    pltpu.sync_copy(obuf, out_hbm.at[wid])


@jax.jit
def _center_loss(xsT, ys2, ctf):
    kfn = pl.kernel(
        _body,
        out_type=jax.ShapeDtypeStruct((NW, L), jnp.float32),
        mesh=plsc.VectorSubcoreMesh(core_axis_name="c", subcore_axis_name="s",
                                    num_cores=2, num_subcores=16),
        compiler_params=pltpu.CompilerParams(use_tc_tiling_on_sc=False),
        scratch_types=[
            pltpu.VMEM((4, 128), jnp.int32),
            pltpu.VMEM((8, 128), jnp.int32),
            pltpu.VMEM((NG, 4, 128), jnp.float32),
            pltpu.VMEM((FEAT, ROWS), jnp.float32),
            pltpu.VMEM((ROWS,), jnp.int32),
            pltpu.VMEM((128,), jnp.int32),
            pltpu.VMEM((4, 128), jnp.int32),
            pltpu.VMEM((L,), jnp.float32),
            pltpu.SemaphoreType.DMA,
            pltpu.SemaphoreType.DMA,
            pltpu.VMEM_SHARED((CLS,), jnp.int32),
        ],
    )
    parts = kfn(xsT, ys2, ctf)
    return jnp.sum(parts) * 0.5


def kernel(xs, ys, center):
    ys2 = ys.astype(jnp.int32).reshape(128, 128)
    ctf = center.T.reshape(FEAT * CLS)
    return _center_loss(xs.T, ys2, ctf)


# speedup vs baseline: 5.0158x; 5.0158x over previous
import functools

import jax
import jax.numpy as jnp
from jax import lax
from jax.experimental import pallas as pl
from jax.experimental.pallas import tpu as pltpu
from jax.experimental.pallas import tpu_sc as plsc

CLS = 1_000_000
FEAT = 32
B = 16384
NW = 32
ROWS = B // NW
L = 16


def _body(xs_hbm, ys_hbm, center_hbm, out_hbm,
          idx2, hys, crows, xsv, cntf, zbuf, obuf, sem, histo):
    cid = lax.axis_index("c")
    sid = lax.axis_index("s")
    wid = sid * 2 + cid

    pltpu.sync_copy(ys_hbm.at[pl.ds(wid * 4, 4)], idx2)
    pltpu.sync_copy(ys_hbm.at[pl.ds(sid * 8, 8)], hys)

    copies = [
        pltpu.async_copy(center_hbm.at[idx2.at[j]],
                         crows.at[pl.ds(j * 128, 128)], sem)
        for j in range(4)
    ]
    copies.append(pltpu.async_copy(xs_hbm.at[pl.ds(wid * ROWS, ROWS)],
                                   xsv, sem))

    for i in range(8):
        zbuf[pl.ds(i * L, L)] = jnp.zeros((L,), jnp.int32)
    for j in range(8):
        pltpu.sync_copy(zbuf, histo.at[hys.at[j]])
    plsc.subcore_barrier()

    for i in range(8):
        zbuf[pl.ds(i * L, L)] = jnp.ones((L,), jnp.int32)
    for j in range(8):
        pltpu.sync_copy(zbuf, histo.at[hys.at[j]], add=True)
    plsc.subcore_barrier()

    for j in range(4):
        pltpu.sync_copy(histo.at[idx2.at[j]], cntf.at[pl.ds(j * 128, 128)])

    for c in copies:
        c.wait()

    def group(g, acc):
        ci = cntf[pl.ds(g * L, L)]
        w = 1.0 / ci.astype(jnp.float32)
        for r in range(L):
            row = g * L + r
            x0 = xsv[row, pl.ds(0, L)]
            x1 = xsv[row, pl.ds(L, L)]
            c0 = crows[row, pl.ds(0, L)]
            c1 = crows[row, pl.ds(L, L)]
            d0 = x0 - c0
            d1 = x1 - c1
            p = d0 * d0 + d1 * d1
            wr = lax.gather(
                w, jnp.full((L, 1), r, jnp.int32),
                dimension_numbers=lax.GatherDimensionNumbers(
                    offset_dims=(), collapsed_slice_dims=(0,),
                    start_index_map=(0,)),
                slice_sizes=(1,),
                mode=lax.GatherScatterMode.PROMISE_IN_BOUNDS)
            acc = acc + p * wr
        return acc

    acc = lax.fori_loop(0, ROWS // L, group, jnp.zeros((L,), jnp.float32))
    obuf[...] = acc
    pltpu.sync_copy(obuf, out_hbm.at[wid])


@jax.jit
def _center_loss(xs, ys2, center):
    kfn = pl.kernel(
        _body,
        out_type=jax.ShapeDtypeStruct((NW, L), jnp.float32),
        mesh=plsc.VectorSubcoreMesh(core_axis_name="c", subcore_axis_name="s",
                                    num_cores=2, num_subcores=16),
        compiler_params=pltpu.CompilerParams(use_tc_tiling_on_sc=False),
        scratch_types=[
            pltpu.VMEM((4, 128), jnp.int32),
            pltpu.VMEM((8, 128), jnp.int32),
            pltpu.VMEM((ROWS, FEAT), jnp.float32),
            pltpu.VMEM((ROWS, FEAT), jnp.float32),
            pltpu.VMEM((ROWS,), jnp.int32),
            pltpu.VMEM((128,), jnp.int32),
            pltpu.VMEM((L,), jnp.float32),
            pltpu.SemaphoreType.DMA,
            pltpu.VMEM_SHARED((CLS,), jnp.int32),
        ],
    )
    parts = kfn(xs, ys2, center)
    return jnp.sum(parts) * 0.5


def kernel(xs, ys, center):
    ys2 = ys.astype(jnp.int32).reshape(128, 128)
    return _center_loss(xs, ys2, center)


# speedup vs baseline: 10.9571x; 2.1845x over previous
import jax
import jax.numpy as jnp
from jax import lax
from jax.experimental import pallas as pl
from jax.experimental.pallas import tpu as pltpu
from jax.experimental.pallas import tpu_sc as plsc

CLS = 1_000_000
FEAT = 32
B = 16384
NW = 32
ROWS = B // NW
L = 16


def _splat(w, k):
    return lax.gather(
        w, jnp.full((L, 1), k, jnp.int32),
        dimension_numbers=lax.GatherDimensionNumbers(
            offset_dims=(), collapsed_slice_dims=(0,), start_index_map=(0,)),
        slice_sizes=(1,),
        mode=lax.GatherScatterMode.PROMISE_IN_BOUNDS)


def _body(w3_hbm, ys_hbm, xs_hbm, out_hbm,
          idx0, idx1, idx2, idx3, hy0, hy1, hy2, hy3, hy4, hy5, hy6, hy7,
          ring, xsv, cntf, zbuf, obuf, sem, hsem, histo):
    cid = lax.axis_index("c")
    sid = lax.axis_index("s")
    wid = sid * 2 + cid
    base = wid * ROWS

    idxs = (idx0, idx1, idx2, idx3)
    hys = (hy0, hy1, hy2, hy3, hy4, hy5, hy6, hy7)

    for j, ref in enumerate(idxs):
        pltpu.sync_copy(ys_hbm.at[pl.ds(base + j * 128, 128)], ref)
    for j, ref in enumerate(hys):
        pltpu.sync_copy(ys_hbm.at[pl.ds(sid * 1024 + j * 128, 128)], ref)

    xcopy = pltpu.async_copy(xs_hbm.at[pl.ds(base * FEAT, ROWS * FEAT)],
                             xsv, hsem)

    for i in range(8):
        zbuf[pl.ds(i * L, L)] = jnp.zeros((L,), jnp.int32)
    for ref in hys:
        pltpu.sync_copy(zbuf, histo.at[ref])
    plsc.subcore_barrier()

    for i in range(8):
        zbuf[pl.ds(i * L, L)] = jnp.ones((L,), jnp.int32)
    for ref in hys:
        pltpu.sync_copy(zbuf, histo.at[ref], add=True)
    plsc.subcore_barrier()

    for j, ref in enumerate(idxs):
        pltpu.sync_copy(histo.at[ref], cntf.at[pl.ds(j * 128, 128)])

    xcopy.wait()

    acc = jnp.zeros((L,), jnp.float32)
    for j in range(4):
        def grp(gg, acc, j=j, idxj=idxs[j]):
            yv = idxj[pl.ds(gg * L, L)]
            copies = []
            for k in range(L):
                t = lax.shift_right_logical(yv[k], 3)
                copies.append(pltpu.async_copy(
                    w3_hbm.at[pl.ds(t, 1)], ring.at[pl.ds(k, 1)], sem))
            for c in copies:
                c.wait()
            ci = cntf[pl.ds(j * 128 + gg * L, L)]
            w = 1.0 / ci.astype(jnp.float32)
            for k in range(L):
                yr = yv[k] & 7
                c0 = ring[k, yr, pl.ds(0, L)]
                c1 = ring[k, yr, pl.ds(L, L)]
                off = (gg * L + k) * FEAT + j * 128 * FEAT
                x0 = xsv[pl.ds(off, L)]
                x1 = xsv[pl.ds(off + L, L)]
                d0 = x0 - c0
                d1 = x1 - c1
                p = d0 * d0 + d1 * d1
                acc = acc + p * _splat(w, k)
            return acc
        acc = lax.fori_loop(0, 8, grp, acc)

    obuf[...] = acc
    pltpu.sync_copy(obuf, out_hbm.at[pl.ds(wid * L, L)])


@jax.jit
def _center_loss(w3, ys1, xsf):
    kfn = pl.kernel(
        _body,
        out_type=jax.ShapeDtypeStruct((NW * L,), jnp.float32),
        mesh=plsc.VectorSubcoreMesh(core_axis_name="c", subcore_axis_name="s",
                                    num_cores=2, num_subcores=16),
        compiler_params=pltpu.CompilerParams(use_tc_tiling_on_sc=True),
        scratch_types=(
            [pltpu.VMEM((128,), jnp.int32) for _ in range(4)] +
            [pltpu.VMEM((128,), jnp.int32) for _ in range(8)] +
            [
                pltpu.VMEM((L, 8, FEAT), jnp.float32),
                pltpu.VMEM((ROWS * FEAT,), jnp.float32),
                pltpu.VMEM((ROWS,), jnp.int32),
                pltpu.VMEM((128,), jnp.int32),
                pltpu.VMEM((L,), jnp.float32),
                pltpu.SemaphoreType.DMA,
                pltpu.SemaphoreType.DMA,
                pltpu.VMEM_SHARED((CLS,), jnp.int32),
            ]
        ),
    )
    parts = kfn(w3, ys1, xsf)
    return jnp.sum(parts) * 0.5


def kernel(xs, ys, center):
    w3 = center.reshape(CLS // 8, 8, FEAT)
    ys1 = ys.astype(jnp.int32)
    xsf = xs.reshape(B * FEAT)
    return _center_loss(w3, ys1, xsf)


# speedup vs baseline: 11.4313x; 1.0433x over previous
import jax
import jax.numpy as jnp
from jax import lax
from jax.experimental import pallas as pl
from jax.experimental.pallas import tpu as pltpu
from jax.experimental.pallas import tpu_sc as plsc

CLS = 1_000_000
FEAT = 32
B = 16384
NW = 32
ROWS = B // NW
L = 16


def _splat(w, k):
    return lax.gather(
        w, jnp.full((L, 1), k, jnp.int32),
        dimension_numbers=lax.GatherDimensionNumbers(
            offset_dims=(), collapsed_slice_dims=(0,), start_index_map=(0,)),
        slice_sizes=(1,),
        mode=lax.GatherScatterMode.PROMISE_IN_BOUNDS)


def _body(w3_hbm, ys_hbm, xs_hbm, out_hbm,
          idx0, idx1, idx2, idx3, hy0, hy1, hy2, hy3, hy4, hy5, hy6, hy7,
          ring, xsv, cntf, zbuf, obuf, sem, hsem, histo):
    cid = lax.axis_index("c")
    sid = lax.axis_index("s")
    wid = sid * 2 + cid
    base = wid * ROWS

    idxs = (idx0, idx1, idx2, idx3)
    hys = (hy0, hy1, hy2, hy3, hy4, hy5, hy6, hy7)

    for j, ref in enumerate(idxs):
        pltpu.sync_copy(ys_hbm.at[pl.ds(base + j * 128, 128)], ref)
    for j, ref in enumerate(hys):
        pltpu.sync_copy(ys_hbm.at[pl.ds(sid * 1024 + j * 128, 128)], ref)

    xcopy = pltpu.async_copy(xs_hbm.at[pl.ds(base * FEAT, ROWS * FEAT)],
                             xsv, hsem)

    for i in range(8):
        zbuf[pl.ds(i * L, L)] = jnp.zeros((L,), jnp.int32)
    for ref in hys:
        pltpu.sync_copy(zbuf, histo.at[ref])
    plsc.subcore_barrier()

    for i in range(8):
        zbuf[pl.ds(i * L, L)] = jnp.ones((L,), jnp.int32)
    for ref in hys:
        pltpu.sync_copy(zbuf, histo.at[ref], add=True)
    plsc.subcore_barrier()

    for j, ref in enumerate(idxs):
        pltpu.sync_copy(histo.at[ref], cntf.at[pl.ds(j * 128, 128)])

    xcopy.wait()

    acc = jnp.zeros((L,), jnp.float32)
    for j in range(4):
        def grp(gg, acc, j=j, idxj=idxs[j]):
            yvs = [idxj[pl.ds(gg * 2 * L, L)], idxj[pl.ds(gg * 2 * L + L, L)]]
            copies = []
            for h, yv in enumerate(yvs):
                for k in range(L):
                    t = lax.shift_right_logical(yv[k], 3)
                    copies.append(pltpu.async_copy(
                        w3_hbm.at[pl.ds(t, 1)],
                        ring.at[pl.ds(h * L + k, 1)], sem))
            for c in copies:
                c.wait()
            for h, yv in enumerate(yvs):
                ci = cntf[pl.ds(j * 128 + (gg * 2 + h) * L, L)]
                w = 1.0 / ci.astype(jnp.float32)
                for k in range(L):
                    yr = yv[k] & 7
                    c0 = ring[h * L + k, yr, pl.ds(0, L)]
                    c1 = ring[h * L + k, yr, pl.ds(L, L)]
                    off = ((gg * 2 + h) * L + k) * FEAT + j * 128 * FEAT
                    x0 = xsv[pl.ds(off, L)]
                    x1 = xsv[pl.ds(off + L, L)]
                    d0 = x0 - c0
                    d1 = x1 - c1
                    p = d0 * d0 + d1 * d1
                    acc = acc + p * _splat(w, k)
            return acc
        acc = lax.fori_loop(0, 4, grp, acc)

    obuf[...] = acc
    pltpu.sync_copy(obuf, out_hbm.at[pl.ds(wid * L, L)])


@jax.jit
def _center_loss(w3, ys1, xsf):
    kfn = pl.kernel(
        _body,
        out_type=jax.ShapeDtypeStruct((NW * L,), jnp.float32),
        mesh=plsc.VectorSubcoreMesh(core_axis_name="c", subcore_axis_name="s",
                                    num_cores=2, num_subcores=16),
        compiler_params=pltpu.CompilerParams(use_tc_tiling_on_sc=True),
        scratch_types=(
            [pltpu.VMEM((128,), jnp.int32) for _ in range(4)] +
            [pltpu.VMEM((128,), jnp.int32) for _ in range(8)] +
            [
                pltpu.VMEM((2 * L, 8, FEAT), jnp.float32),
                pltpu.VMEM((ROWS * FEAT,), jnp.float32),
                pltpu.VMEM((ROWS,), jnp.int32),
                pltpu.VMEM((128,), jnp.int32),
                pltpu.VMEM((L,), jnp.float32),
                pltpu.SemaphoreType.DMA,
                pltpu.SemaphoreType.DMA,
                pltpu.VMEM_SHARED((CLS,), jnp.int32),
            ]
        ),
    )
    parts = kfn(w3, ys1, xsf)
    return jnp.sum(parts) * 0.5


def kernel(xs, ys, center):
    w3 = center.reshape(CLS // 8, 8, FEAT)
    ys1 = ys.astype(jnp.int32)
    xsf = xs.reshape(B * FEAT)
    return _center_loss(w3, ys1, xsf)
